# Initial kernel scaffold; baseline (speedup 1.0000x reference)
import jax
import jax.numpy as jnp
from jax.experimental import pallas as pl


def kernel(x, w1, gamma, beta, mean, var, w2, b2):
    raise NotImplementedError("write your pallas kernel here")


# speedup vs baseline: 237.2736x; 237.2736x over previous
import jax
import jax.numpy as jnp
import numpy as np
from jax.experimental import pallas as pl
from jax.experimental.pallas import tpu as pltpu

B, CIN, COUT = 8, 128, 32
H = W = 128
HW = H * W
A, R = 180, 184
EPS = 1e-5
NR = 4
PB = NR * W
PC = 2048
NC = HW // PC


def _rho_table():
    theta = jnp.arange(A, dtype=jnp.float32) * (np.pi / A)
    ii, jj = jnp.meshgrid(jnp.arange(H, dtype=jnp.float32),
                          jnp.arange(W, dtype=jnp.float32), indexing='ij')
    xc = jj - (W - 1) / 2.0
    yc = ii - (H - 1) / 2.0
    irho = np.sqrt(H * H + W * W) / R
    rho = (xc[None] * jnp.cos(theta)[:, None, None]
           + yc[None] * jnp.sin(theta)[:, None, None])
    r = jnp.round(rho / irho).astype(jnp.int32) + R // 2
    return jnp.clip(r, 0, R - 1).reshape(A, 1, HW)


def _conv_body(xp_ref, w1_ref, sc_ref, bi_ref, w2_ref, b2_ref, out_ref, om_ref):
    Lb = (NR + 2) * W
    jmod = jax.lax.broadcasted_iota(jnp.int32, (1, Lb), 1) % W
    scale = sc_ref[...]
    bias = bi_ref[...]
    b2b = b2_ref[...]
    w2 = w2_ref[...]
    for rb in range(H // NR):
        base = xp_ref[0, :, rb * NR * W:(rb * NR + NR + 2) * W]
        sh_m = jnp.where(
            jmod == 0, 0.0,
            jnp.concatenate([jnp.zeros_like(base[:, :1]), base[:, :-1]], axis=1))
        sh_p = jnp.where(
            jmod == W - 1, 0.0,
            jnp.concatenate([base[:, 1:], jnp.zeros_like(base[:, :1])], axis=1))
        acc = None
        for di in range(3):
            for dj, src in ((0, sh_m), (1, base), (2, sh_p)):
                xs = src[:, di * W:di * W + PB]
                d = jnp.dot(w1_ref[di * 3 + dj], xs,
                            preferred_element_type=jnp.float32)
                acc = d if acc is None else acc + d
        h = jnp.maximum(acc * scale + bias, 0.0)
        o = jnp.dot(w2, h, preferred_element_type=jnp.float32) + b2b
        sl = slice(rb * PB, (rb + 1) * PB)
        out_ref[0, :, sl] = o
        om_ref[0, :, sl] = jnp.sum(o, axis=0, keepdims=True) * (1.0 / COUT)


def _hough_body(rf_ref, om_ref, mp_ref, oh_ref):
    a = pl.program_id(0)

    @pl.when(a == 0)
    def _init():
        mp_ref[...] = jnp.zeros_like(mp_ref)

    om = om_ref[...].reshape(B, HW)
    rf = rf_ref[...].reshape(1, HW)
    hm = None
    for c in range(NC):
        sl = slice(c * PC, (c + 1) * PC)
        iot = jax.lax.broadcasted_iota(jnp.int32, (R, PC), 0)
        oh = jnp.where(iot == rf[:, sl], 1.0, 0.0)
        oh_ref[:, sl] = oh
        part = jax.lax.dot_general(om[:, sl], oh, (((1,), (1,)), ((), ())),
                                   preferred_element_type=jnp.float32)
        hm = part if hm is None else hm + part
    for c in range(NC):
        sl = slice(c * PC, (c + 1) * PC)
        oh = oh_ref[:, sl]
        mp_ref[:, 0, sl] += jnp.dot(hm, oh, preferred_element_type=jnp.float32)


def _mask_body(mp_ref, mask_ref):
    v = mp_ref[...].reshape(1, HW)
    s = jax.nn.sigmoid(v)
    mask_ref[0] = jnp.broadcast_to(s, (COUT, HW))


def kernel(x, w1, gamma, beta, mean, var, w2, b2):
    xpad = jnp.pad(x, ((0, 0), (0, 0), (1, 1), (0, 0))).reshape(B, CIN, (H + 2) * W)
    w1r = jnp.transpose(w1, (2, 3, 0, 1)).reshape(9, CIN, CIN)
    inv = gamma / jnp.sqrt(var + EPS)
    sc = jnp.broadcast_to(inv[:, None], (CIN, PB))
    bi = jnp.broadcast_to((beta - mean * inv)[:, None], (CIN, PB))
    w2r = w2.reshape(COUT, CIN)
    b2r = jnp.broadcast_to(b2[:, None], (COUT, PB))
    rf = _rho_table()

    out, om = pl.pallas_call(
        _conv_body,
        grid=(B,),
        in_specs=[
            pl.BlockSpec((1, CIN, (H + 2) * W), lambda b: (b, 0, 0)),
            pl.BlockSpec((9, CIN, CIN), lambda b: (0, 0, 0)),
            pl.BlockSpec((CIN, PB), lambda b: (0, 0)),
            pl.BlockSpec((CIN, PB), lambda b: (0, 0)),
            pl.BlockSpec((COUT, CIN), lambda b: (0, 0)),
            pl.BlockSpec((COUT, PB), lambda b: (0, 0)),
        ],
        out_specs=[
            pl.BlockSpec((1, COUT, HW), lambda b: (b, 0, 0)),
            pl.BlockSpec((1, 1, HW), lambda b: (b, 0, 0)),
        ],
        out_shape=[
            jax.ShapeDtypeStruct((B, COUT, HW), jnp.float32),
            jax.ShapeDtypeStruct((B, 1, HW), jnp.float32),
        ],
        compiler_params=pltpu.CompilerParams(
            dimension_semantics=("arbitrary",),
            vmem_limit_bytes=60 * 1024 * 1024,
        ),
        name="conv_fused",
    )(xpad, w1r, sc, bi, w2r, b2r)

    mask_pre = pl.pallas_call(
        _hough_body,
        grid=(A,),
        in_specs=[
            pl.BlockSpec((1, 1, HW), lambda a: (a, 0, 0)),
            pl.BlockSpec((B, 1, HW), lambda a: (0, 0, 0)),
        ],
        out_specs=pl.BlockSpec((B, 1, HW), lambda a: (0, 0, 0)),
        out_shape=jax.ShapeDtypeStruct((B, 1, HW), jnp.float32),
        scratch_shapes=[pltpu.VMEM((R, HW), jnp.float32)],
        compiler_params=pltpu.CompilerParams(
            dimension_semantics=("arbitrary",),
            vmem_limit_bytes=60 * 1024 * 1024,
        ),
        name="hough_vote_backproject",
    )(rf, om)

    mask = pl.pallas_call(
        _mask_body,
        grid=(B,),
        in_specs=[pl.BlockSpec((1, 1, HW), lambda b: (b, 0, 0))],
        out_specs=pl.BlockSpec((1, COUT, HW), lambda b: (b, 0, 0)),
        out_shape=jax.ShapeDtypeStruct((B, COUT, HW), jnp.float32),
        compiler_params=pltpu.CompilerParams(
            dimension_semantics=("arbitrary",),
        ),
        name="mask_sigmoid",
    )(mask_pre)

    return mask.reshape(B, COUT, H, W), out.reshape(B, COUT, H, W)


# speedup vs baseline: 239.7158x; 1.0103x over previous
import jax
import jax.numpy as jnp
import numpy as np
from jax.experimental import pallas as pl
from jax.experimental.pallas import tpu as pltpu

B, CIN, COUT = 8, 128, 32
H = W = 128
HW = H * W
A, R = 180, 184
EPS = 1e-5
NR = 4
PB = NR * W
PC = 2048
NC = HW // PC
NCORE = 2
AC = A // NCORE


def _rho_table():
    theta = jnp.arange(A, dtype=jnp.float32) * (np.pi / A)
    ii, jj = jnp.meshgrid(jnp.arange(H, dtype=jnp.float32),
                          jnp.arange(W, dtype=jnp.float32), indexing='ij')
    xc = jj - (W - 1) / 2.0
    yc = ii - (H - 1) / 2.0
    irho = np.sqrt(H * H + W * W) / R
    rho = (xc[None] * jnp.cos(theta)[:, None, None]
           + yc[None] * jnp.sin(theta)[:, None, None])
    r = jnp.round(rho / irho).astype(jnp.int32) + R // 2
    return jnp.clip(r, 0, R - 1).reshape(A, 1, HW)


def _conv_body(x_ref, w1_ref, sc_ref, bi_ref, w2_ref, b2_ref, out_ref, om_ref):
    Lb = (NR + 2) * W
    jmod = jax.lax.broadcasted_iota(jnp.int32, (1, Lb), 1) % W
    scale = sc_ref[...]
    bias = bi_ref[...]
    b2b = b2_ref[...]
    w2 = w2_ref[...]
    zrow = jnp.zeros((CIN, W), jnp.bfloat16)
    nblk = H // NR
    for rb in range(nblk):
        lo = rb * NR - 1
        if rb == 0:
            base = jnp.concatenate(
                [zrow, x_ref[0, :, 0:(NR + 1) * W]], axis=1)
        elif rb == nblk - 1:
            base = jnp.concatenate(
                [x_ref[0, :, lo * W:HW], zrow], axis=1)
        else:
            base = x_ref[0, :, lo * W:(lo + NR + 2) * W]
        sh_m = jnp.where(
            jmod == 0, jnp.bfloat16(0),
            jnp.concatenate([zrow[:, :1], base[:, :-1]], axis=1))
        sh_p = jnp.where(
            jmod == W - 1, jnp.bfloat16(0),
            jnp.concatenate([base[:, 1:], zrow[:, :1]], axis=1))
        acc = None
        for di in range(3):
            sl = slice(di * W, di * W + PB)
            xcat = jnp.concatenate(
                [sh_m[:, sl], base[:, sl], sh_p[:, sl]], axis=0)
            d = jnp.dot(w1_ref[di], xcat, preferred_element_type=jnp.float32)
            acc = d if acc is None else acc + d
        h = jnp.maximum(acc * scale + bias, 0.0).astype(jnp.bfloat16)
        o = jnp.dot(w2, h, preferred_element_type=jnp.float32) + b2b
        sl = slice(rb * PB, (rb + 1) * PB)
        out_ref[0, :, sl] = o
        om_ref[0, :, sl] = (jnp.sum(o, axis=0, keepdims=True)
                            * (1.0 / COUT)).astype(jnp.bfloat16)


def _hough_body(rf_ref, om_ref, mp_ref, oh_ref):
    s = pl.program_id(1)

    @pl.when(s == 0)
    def _init():
        mp_ref[...] = jnp.zeros_like(mp_ref)

    om = om_ref[...].reshape(B, HW)
    rf = rf_ref[...].reshape(1, HW)
    one = jnp.bfloat16(1)
    zero = jnp.bfloat16(0)
    iot = jax.lax.broadcasted_iota(
        jnp.int32, (R, PC), 0).astype(jnp.bfloat16)
    hm = None
    for c in range(NC):
        sl = slice(c * PC, (c + 1) * PC)
        rfb = jnp.broadcast_to(rf[:, sl], (R, PC))
        oh = jnp.where(iot == rfb, one, zero)
        oh_ref[:, sl] = oh
        part = jax.lax.dot_general(om[:, sl], oh, (((1,), (1,)), ((), ())),
                                   preferred_element_type=jnp.float32)
        hm = part if hm is None else hm + part
    hmb = hm.astype(jnp.bfloat16)
    for c in range(NC):
        sl = slice(c * PC, (c + 1) * PC)
        oh = oh_ref[:, sl]
        mp_ref[0, :, sl] += jnp.dot(hmb, oh, preferred_element_type=jnp.float32)


def _mask_body(mp_ref, mask_ref):
    v = (mp_ref[0, 0] + mp_ref[1, 0]).reshape(1, HW)
    sg = jax.nn.sigmoid(v)
    mask_ref[0] = jnp.broadcast_to(sg, (COUT, HW))


def kernel(x, w1, gamma, beta, mean, var, w2, b2):
    xb = x.reshape(B, CIN, HW).astype(jnp.bfloat16)
    w1r = jnp.transpose(w1, (2, 0, 3, 1)).reshape(3, CIN, 3 * CIN)
    w1r = w1r.astype(jnp.bfloat16)
    inv = gamma / jnp.sqrt(var + EPS)
    sc = jnp.broadcast_to(inv[:, None], (CIN, PB))
    bi = jnp.broadcast_to((beta - mean * inv)[:, None], (CIN, PB))
    w2r = w2.reshape(COUT, CIN).astype(jnp.bfloat16)
    b2r = jnp.broadcast_to(b2[:, None], (COUT, PB))
    rf = _rho_table().astype(jnp.bfloat16).reshape(NCORE, AC, 1, HW)

    out, om = pl.pallas_call(
        _conv_body,
        grid=(NCORE, B // NCORE),
        in_specs=[
            pl.BlockSpec((1, CIN, HW), lambda c, i: (c * (B // NCORE) + i, 0, 0)),
            pl.BlockSpec((3, CIN, 3 * CIN), lambda c, i: (0, 0, 0)),
            pl.BlockSpec((CIN, PB), lambda c, i: (0, 0)),
            pl.BlockSpec((CIN, PB), lambda c, i: (0, 0)),
            pl.BlockSpec((COUT, CIN), lambda c, i: (0, 0)),
            pl.BlockSpec((COUT, PB), lambda c, i: (0, 0)),
        ],
        out_specs=[
            pl.BlockSpec((1, COUT, HW), lambda c, i: (c * (B // NCORE) + i, 0, 0)),
            pl.BlockSpec((1, 1, HW), lambda c, i: (c * (B // NCORE) + i, 0, 0)),
        ],
        out_shape=[
            jax.ShapeDtypeStruct((B, COUT, HW), jnp.float32),
            jax.ShapeDtypeStruct((B, 1, HW), jnp.bfloat16),
        ],
        compiler_params=pltpu.CompilerParams(
            dimension_semantics=("arbitrary", "arbitrary"),
            vmem_limit_bytes=60 * 1024 * 1024,
        ),
        name="conv_fused",
    )(xb, w1r, sc, bi, w2r, b2r)

    mask_parts = pl.pallas_call(
        _hough_body,
        grid=(NCORE, AC),
        in_specs=[
            pl.BlockSpec((1, 1, 1, HW), lambda c, s: (c, s, 0, 0)),
            pl.BlockSpec((B, 1, HW), lambda c, s: (0, 0, 0)),
        ],
        out_specs=pl.BlockSpec((1, B, HW), lambda c, s: (c, 0, 0)),
        out_shape=jax.ShapeDtypeStruct((NCORE, B, HW), jnp.float32),
        scratch_shapes=[pltpu.VMEM((R, HW), jnp.bfloat16)],
        compiler_params=pltpu.CompilerParams(
            dimension_semantics=("arbitrary", "arbitrary"),
            vmem_limit_bytes=60 * 1024 * 1024,
        ),
        name="hough_vote_backproject",
    )(rf, om)

    mask = pl.pallas_call(
        _mask_body,
        grid=(NCORE, B // NCORE),
        in_specs=[pl.BlockSpec((NCORE, 1, 1, HW),
                               lambda c, i: (0, c * (B // NCORE) + i, 0, 0))],
        out_specs=pl.BlockSpec((1, COUT, HW),
                               lambda c, i: (c * (B // NCORE) + i, 0, 0)),
        out_shape=jax.ShapeDtypeStruct((B, COUT, HW), jnp.float32),
        compiler_params=pltpu.CompilerParams(
            dimension_semantics=("arbitrary", "arbitrary"),
        ),
        name="mask_sigmoid",
    )(mask_parts.reshape(NCORE, B, 1, HW))

    return mask.reshape(B, COUT, H, W), out.reshape(B, COUT, H, W)


# speedup vs baseline: 405.3068x; 1.6908x over previous
import jax
import jax.numpy as jnp
import numpy as np
from jax.experimental import pallas as pl
from jax.experimental.pallas import tpu as pltpu

B, CIN, COUT = 8, 128, 32
H = W = 128
HW = H * W
A, R = 180, 184
EPS = 1e-5
NR = 4
PB = NR * W
PC = 2048
NC = HW // PC
NCORE = 2
GA = 2
NB = 2
APS = GA * NB
NSTEP = A // APS
RG = R * GA


def _rho_table():
    theta = jnp.arange(A, dtype=jnp.float32) * (np.pi / A)
    ii, jj = jnp.meshgrid(jnp.arange(H, dtype=jnp.float32),
                          jnp.arange(W, dtype=jnp.float32), indexing='ij')
    xc = jj - (W - 1) / 2.0
    yc = ii - (H - 1) / 2.0
    irho = np.sqrt(H * H + W * W) / R
    rho = (xc[None] * jnp.cos(theta)[:, None, None]
           + yc[None] * jnp.sin(theta)[:, None, None])
    r = jnp.round(rho / irho).astype(jnp.int32) + R // 2
    return jnp.clip(r, 0, R - 1).reshape(A, 1, HW)


def _conv_body(x_ref, w1_ref, sc_ref, bi_ref, w2_ref, b2_ref, out_ref, om_ref):
    Lb = (NR + 2) * W
    jmod = jax.lax.broadcasted_iota(jnp.int32, (1, Lb), 1) % W
    scale = sc_ref[...]
    bias = bi_ref[...]
    b2b = b2_ref[...]
    w2 = w2_ref[...]
    zrow = jnp.zeros((CIN, W), jnp.bfloat16)
    nblk = H // NR
    for rb in range(nblk):
        lo = rb * NR - 1
        if rb == 0:
            base = jnp.concatenate(
                [zrow, x_ref[0, :, 0:(NR + 1) * W]], axis=1)
        elif rb == nblk - 1:
            base = jnp.concatenate(
                [x_ref[0, :, lo * W:HW], zrow], axis=1)
        else:
            base = x_ref[0, :, lo * W:(lo + NR + 2) * W]
        sh_m = jnp.where(
            jmod == 0, jnp.bfloat16(0),
            jnp.concatenate([zrow[:, :1], base[:, :-1]], axis=1))
        sh_p = jnp.where(
            jmod == W - 1, jnp.bfloat16(0),
            jnp.concatenate([base[:, 1:], zrow[:, :1]], axis=1))
        acc = None
        for di in range(3):
            sl = slice(di * W, di * W + PB)
            xcat = jnp.concatenate(
                [sh_m[:, sl], base[:, sl], sh_p[:, sl]], axis=0)
            d = jnp.dot(w1_ref[di], xcat, preferred_element_type=jnp.float32)
            acc = d if acc is None else acc + d
        h = jnp.maximum(acc * scale + bias, 0.0).astype(jnp.bfloat16)
        o = jnp.dot(w2, h, preferred_element_type=jnp.float32) + b2b
        sl = slice(rb * PB, (rb + 1) * PB)
        out_ref[0, :, sl] = o
        om_ref[0, :, sl] = (jnp.sum(o, axis=0, keepdims=True)
                            * (1.0 / COUT)).astype(jnp.bfloat16)


def _hough_body(rfb_ref, rfi_ref, om_ref, mp_ref):
    s = pl.program_id(0)

    @pl.when(s == 0)
    def _init():
        mp_ref[...] = jnp.zeros_like(mp_ref)

    one = jnp.bfloat16(1)
    zero = jnp.bfloat16(0)
    iotg = (jax.lax.broadcasted_iota(jnp.int32, (RG, PC), 0) % R
            ).astype(jnp.bfloat16)
    omb = [om_ref[:, 0, c * PC:(c + 1) * PC].astype(jnp.bfloat16)
           for c in range(NC)]
    hms = []
    for b in range(NB):
        rfp = rfb_ref[0, 2 * b:2 * b + 2, :]
        acc = None
        for c in range(NC):
            sl = slice(c * PC, (c + 1) * PC)
            rfcat = jnp.concatenate(
                [jnp.broadcast_to(rfp[g:g + 1, sl], (R, PC)) for g in range(GA)],
                axis=0)
            oh = jnp.where(iotg == rfcat, one, zero)
            part = jax.lax.dot_general(omb[c], oh, (((1,), (1,)), ((), ())),
                                       preferred_element_type=jnp.float32)
            acc = part if acc is None else acc + part
        hms.append(acc)
    gsum = None
    for b in range(NB):
        for g in range(GA):
            hm_g = hms[b][:, g * R:(g + 1) * R]
            idx = jnp.broadcast_to(
                rfi_ref[0, 2 * b + g:2 * b + g + 1, :], (B, HW))
            lo = jnp.take_along_axis(hm_g[:, :128], idx, axis=1)
            hi = jnp.take_along_axis(hm_g[:, 128:R], idx - 128, axis=1)
            gth = jnp.where(idx < 128, lo, hi)
            gsum = gth if gsum is None else gsum + gth
    mp_ref[:, 0, :] += gsum


def _mask_body(mp_ref, mask_ref):
    v = mp_ref[...].reshape(1, HW)
    sg = jax.nn.sigmoid(v)
    mask_ref[0] = jnp.broadcast_to(sg, (COUT, HW))


def kernel(x, w1, gamma, beta, mean, var, w2, b2):
    xb = x.reshape(B, CIN, HW).astype(jnp.bfloat16)
    w1r = jnp.transpose(w1, (2, 0, 3, 1)).reshape(3, CIN, 3 * CIN)
    w1r = w1r.astype(jnp.bfloat16)
    inv = gamma / jnp.sqrt(var + EPS)
    sc = jnp.broadcast_to(inv[:, None], (CIN, PB))
    bi = jnp.broadcast_to((beta - mean * inv)[:, None], (CIN, PB))
    w2r = w2.reshape(COUT, CIN).astype(jnp.bfloat16)
    b2r = jnp.broadcast_to(b2[:, None], (COUT, PB))
    rfi = _rho_table().reshape(NSTEP, APS, HW)
    rfb = rfi.astype(jnp.bfloat16)

    out, om = pl.pallas_call(
        _conv_body,
        grid=(NCORE, B // NCORE),
        in_specs=[
            pl.BlockSpec((1, CIN, HW), lambda c, i: (c * (B // NCORE) + i, 0, 0)),
            pl.BlockSpec((3, CIN, 3 * CIN), lambda c, i: (0, 0, 0)),
            pl.BlockSpec((CIN, PB), lambda c, i: (0, 0)),
            pl.BlockSpec((CIN, PB), lambda c, i: (0, 0)),
            pl.BlockSpec((COUT, CIN), lambda c, i: (0, 0)),
            pl.BlockSpec((COUT, PB), lambda c, i: (0, 0)),
        ],
        out_specs=[
            pl.BlockSpec((1, COUT, HW), lambda c, i: (c * (B // NCORE) + i, 0, 0)),
            pl.BlockSpec((1, 1, HW), lambda c, i: (c * (B // NCORE) + i, 0, 0)),
        ],
        out_shape=[
            jax.ShapeDtypeStruct((B, COUT, HW), jnp.float32),
            jax.ShapeDtypeStruct((B, 1, HW), jnp.bfloat16),
        ],
        compiler_params=pltpu.CompilerParams(
            dimension_semantics=("arbitrary", "arbitrary"),
            vmem_limit_bytes=60 * 1024 * 1024,
        ),
        name="conv_fused",
    )(xb, w1r, sc, bi, w2r, b2r)

    mask_pre = pl.pallas_call(
        _hough_body,
        grid=(NSTEP,),
        in_specs=[
            pl.BlockSpec((1, APS, HW), lambda s: (s, 0, 0)),
            pl.BlockSpec((1, APS, HW), lambda s: (s, 0, 0)),
            pl.BlockSpec((B, 1, HW), lambda s: (0, 0, 0)),
        ],
        out_specs=pl.BlockSpec((B, 1, HW), lambda s: (0, 0, 0)),
        out_shape=jax.ShapeDtypeStruct((B, 1, HW), jnp.float32),
        compiler_params=pltpu.CompilerParams(
            dimension_semantics=("arbitrary",),
            vmem_limit_bytes=60 * 1024 * 1024,
        ),
        name="hough_vote_backproject",
    )(rfb, rfi, om)

    mask = pl.pallas_call(
        _mask_body,
        grid=(B,),
        in_specs=[pl.BlockSpec((1, 1, HW), lambda b: (b, 0, 0))],
        out_specs=pl.BlockSpec((1, COUT, HW), lambda b: (b, 0, 0)),
        out_shape=jax.ShapeDtypeStruct((B, COUT, HW), jnp.float32),
        compiler_params=pltpu.CompilerParams(
            dimension_semantics=("arbitrary",),
        ),
        name="mask_sigmoid",
    )(mask_pre)

    return mask.reshape(B, COUT, H, W), out.reshape(B, COUT, H, W)


# speedup vs baseline: 425.0622x; 1.0487x over previous
import jax
import jax.numpy as jnp
import ml_dtypes
import numpy as np
from jax.experimental import pallas as pl
from jax.experimental.pallas import tpu as pltpu

B, CIN, COUT = 8, 128, 32
H = W = 128
HW = H * W
A, R = 180, 184
EPS = 1e-5
NR = 4
PB = NR * W
PC = 2048
NC = HW // PC
NCORE = 2
GA = 2
NB = 2
APS = GA * NB
NSTEP = A // APS
RG = R * GA


def _rho_table_np():
    theta = np.arange(A, dtype=np.float32) * np.float32(np.pi / A)
    ii, jj = np.meshgrid(np.arange(H, dtype=np.float32),
                         np.arange(W, dtype=np.float32), indexing='ij')
    xc = (jj - np.float32((W - 1) / 2.0)).astype(np.float32)
    yc = (ii - np.float32((H - 1) / 2.0)).astype(np.float32)
    irho = np.float32(np.sqrt(H * H + W * W) / R)
    rho = (xc[None] * np.cos(theta, dtype=np.float32)[:, None, None]
           + yc[None] * np.sin(theta, dtype=np.float32)[:, None, None])
    r = np.round(rho / irho).astype(np.int32) + R // 2
    return np.clip(r, 0, R - 1).reshape(NSTEP, APS, HW)


_RFI_NP = _rho_table_np()
_RFB_NP = _RFI_NP.astype(ml_dtypes.bfloat16)


def _conv_body(x_ref, w1_ref, sc_ref, bi_ref, w2_ref, b2_ref, out_ref, om_ref):
    Lb = (NR + 2) * W
    jmod = jax.lax.broadcasted_iota(jnp.int32, (1, Lb), 1) % W
    scale = sc_ref[...]
    bias = bi_ref[...]
    b2b = b2_ref[...]
    w2 = w2_ref[...]
    zrow = jnp.zeros((CIN, W), jnp.bfloat16)
    nblk = H // NR
    for rb in range(nblk):
        lo = rb * NR - 1
        if rb == 0:
            base = jnp.concatenate(
                [zrow, x_ref[0, :, 0:(NR + 1) * W].astype(jnp.bfloat16)], axis=1)
        elif rb == nblk - 1:
            base = jnp.concatenate(
                [x_ref[0, :, lo * W:HW].astype(jnp.bfloat16), zrow], axis=1)
        else:
            base = x_ref[0, :, lo * W:(lo + NR + 2) * W].astype(jnp.bfloat16)
        sh_m = jnp.where(
            jmod == 0, jnp.bfloat16(0),
            jnp.concatenate([zrow[:, :1], base[:, :-1]], axis=1))
        sh_p = jnp.where(
            jmod == W - 1, jnp.bfloat16(0),
            jnp.concatenate([base[:, 1:], zrow[:, :1]], axis=1))
        acc = None
        for di in range(3):
            sl = slice(di * W, di * W + PB)
            xcat = jnp.concatenate(
                [sh_m[:, sl], base[:, sl], sh_p[:, sl]], axis=0)
            d = jnp.dot(w1_ref[di], xcat, preferred_element_type=jnp.float32)
            acc = d if acc is None else acc + d
        h = jnp.maximum(acc * scale + bias, 0.0).astype(jnp.bfloat16)
        o = jnp.dot(w2, h, preferred_element_type=jnp.float32) + b2b
        sl = slice(rb * PB, (rb + 1) * PB)
        out_ref[0, :, sl] = o
        om_ref[0, :, sl] = (jnp.sum(o, axis=0, keepdims=True)
                            * (1.0 / COUT)).astype(jnp.bfloat16)


def _hough_body(rfb_ref, rfi_ref, om_ref, mp_ref):
    s = pl.program_id(0)

    @pl.when(s == 0)
    def _init():
        mp_ref[...] = jnp.zeros_like(mp_ref)

    one = jnp.bfloat16(1)
    zero = jnp.bfloat16(0)
    iotg = (jax.lax.broadcasted_iota(jnp.int32, (RG, PC), 0) % R
            ).astype(jnp.bfloat16)
    omb = [om_ref[:, 0, c * PC:(c + 1) * PC].astype(jnp.bfloat16)
           for c in range(NC)]
    hms = []
    for b in range(NB):
        rfp = rfb_ref[0, 2 * b:2 * b + 2, :]
        acc = None
        for c in range(NC):
            sl = slice(c * PC, (c + 1) * PC)
            rfcat = jnp.concatenate(
                [jnp.broadcast_to(rfp[g:g + 1, sl], (R, PC)) for g in range(GA)],
                axis=0)
            oh = jnp.where(iotg == rfcat, one, zero)
            part = jax.lax.dot_general(omb[c], oh, (((1,), (1,)), ((), ())),
                                       preferred_element_type=jnp.float32)
            acc = part if acc is None else acc + part
        hms.append(acc)
    gsum = None
    for b in range(NB):
        for g in range(GA):
            hm_g = hms[b][:, g * R:(g + 1) * R]
            idx = jnp.broadcast_to(
                rfi_ref[0, 2 * b + g:2 * b + g + 1, :], (B, HW))
            lo = jnp.take_along_axis(hm_g[:, :128], idx, axis=1)
            hi = jnp.take_along_axis(hm_g[:, 128:R], idx - 128, axis=1)
            gth = jnp.where(idx < 128, lo, hi)
            gsum = gth if gsum is None else gsum + gth
    mp_ref[:, 0, :] += gsum


def _mask_body(mp_ref, mask_ref):
    v = mp_ref[...].reshape(1, HW)
    sg = jax.nn.sigmoid(v)
    mask_ref[0] = jnp.broadcast_to(sg, (COUT, HW))


def kernel(x, w1, gamma, beta, mean, var, w2, b2):
    xb = x.reshape(B, CIN, HW)
    w1r = jnp.transpose(w1, (2, 0, 3, 1)).reshape(3, CIN, 3 * CIN)
    w1r = w1r.astype(jnp.bfloat16)
    inv = gamma / jnp.sqrt(var + EPS)
    sc = jnp.broadcast_to(inv[:, None], (CIN, PB))
    bi = jnp.broadcast_to((beta - mean * inv)[:, None], (CIN, PB))
    w2r = w2.reshape(COUT, CIN).astype(jnp.bfloat16)
    b2r = jnp.broadcast_to(b2[:, None], (COUT, PB))
    rfi = jnp.asarray(_RFI_NP)
    rfb = jnp.asarray(_RFB_NP)

    out, om = pl.pallas_call(
        _conv_body,
        grid=(NCORE, B // NCORE),
        in_specs=[
            pl.BlockSpec((1, CIN, HW), lambda c, i: (c * (B // NCORE) + i, 0, 0)),
            pl.BlockSpec((3, CIN, 3 * CIN), lambda c, i: (0, 0, 0)),
            pl.BlockSpec((CIN, PB), lambda c, i: (0, 0)),
            pl.BlockSpec((CIN, PB), lambda c, i: (0, 0)),
            pl.BlockSpec((COUT, CIN), lambda c, i: (0, 0)),
            pl.BlockSpec((COUT, PB), lambda c, i: (0, 0)),
        ],
        out_specs=[
            pl.BlockSpec((1, COUT, HW), lambda c, i: (c * (B // NCORE) + i, 0, 0)),
            pl.BlockSpec((1, 1, HW), lambda c, i: (c * (B // NCORE) + i, 0, 0)),
        ],
        out_shape=[
            jax.ShapeDtypeStruct((B, COUT, HW), jnp.float32),
            jax.ShapeDtypeStruct((B, 1, HW), jnp.bfloat16),
        ],
        compiler_params=pltpu.CompilerParams(
            dimension_semantics=("arbitrary", "arbitrary"),
            vmem_limit_bytes=60 * 1024 * 1024,
        ),
        name="conv_fused",
    )(xb, w1r, sc, bi, w2r, b2r)

    mask_pre = pl.pallas_call(
        _hough_body,
        grid=(NSTEP,),
        in_specs=[
            pl.BlockSpec((1, APS, HW), lambda s: (s, 0, 0)),
            pl.BlockSpec((1, APS, HW), lambda s: (s, 0, 0)),
            pl.BlockSpec((B, 1, HW), lambda s: (0, 0, 0)),
        ],
        out_specs=pl.BlockSpec((B, 1, HW), lambda s: (0, 0, 0)),
        out_shape=jax.ShapeDtypeStruct((B, 1, HW), jnp.float32),
        compiler_params=pltpu.CompilerParams(
            dimension_semantics=("arbitrary",),
            vmem_limit_bytes=60 * 1024 * 1024,
        ),
        name="hough_vote_backproject",
    )(rfb, rfi, om)

    mask = pl.pallas_call(
        _mask_body,
        grid=(B,),
        in_specs=[pl.BlockSpec((1, 1, HW), lambda b: (b, 0, 0))],
        out_specs=pl.BlockSpec((1, COUT, HW), lambda b: (b, 0, 0)),
        out_shape=jax.ShapeDtypeStruct((B, COUT, HW), jnp.float32),
        compiler_params=pltpu.CompilerParams(
            dimension_semantics=("arbitrary",),
        ),
        name="mask_sigmoid",
    )(mask_pre)

    return mask.reshape(B, COUT, H, W), out.reshape(B, COUT, H, W)
